# baseline (speedup 1.0000x reference)
import functools

import jax
import jax.numpy as jnp
from jax import lax
from jax.experimental import pallas as pl
from jax.experimental.pallas import tpu as pltpu
from jax.experimental.pallas import tpu_sc as plsc

NBUF = 7
CHUNK = 16


@jax.jit
def _gather_rows_sc(time2d, pe):
    bsz, seq = time2d.shape
    d = pe.shape[1]
    n = bsz * seq
    info = plsc.get_sparse_core_info()
    num_cores = info.num_cores
    nw = num_cores * info.num_subcores
    n_per_w = n // nw
    n_ch = n_per_w // CHUNK
    w_per_b = seq // n_per_w
    assert n_per_w * nw == n and n_ch * CHUNK == n_per_w
    assert w_per_b * n_per_w == seq

    mesh = plsc.VectorSubcoreMesh(core_axis_name="c", subcore_axis_name="s")

    @functools.partial(
        pl.kernel,
        mesh=mesh,
        out_type=jax.ShapeDtypeStruct((bsz, seq, d), jnp.float32),
        scratch_types=[
            pltpu.VMEM((n_per_w,), jnp.int32),
            pltpu.VMEM((NBUF, CHUNK, d), jnp.float32),
            pltpu.SemaphoreType.DMA((NBUF,)),
        ],
    )
    def k(idx_hbm, pe_hbm, out_hbm, idx_v, rows_v, sems):
        wid = lax.axis_index("s") * num_cores + lax.axis_index("c")
        b = wid // w_per_b
        col = (wid % w_per_b) * n_per_w
        pltpu.sync_copy(idx_hbm.at[b, pl.ds(col, n_per_w)], idx_v)

        def start_gather(c):
            bf = c % NBUF
            return pltpu.async_copy(
                pe_hbm.at[idx_v.at[pl.ds(c * CHUNK, CHUNK)]],
                rows_v.at[bf],
                sems.at[bf],
            )

        def start_store(c):
            bf = c % NBUF
            return pltpu.async_copy(
                rows_v.at[bf],
                out_hbm.at[b, pl.ds(col + c * CHUNK, CHUNK)],
                sems.at[bf],
            )

        gathers = {}
        stores = {}
        for c in range(min(NBUF - 1, n_ch)):
            gathers[c] = start_gather(c)
        for c in range(n_ch):
            nxt = c + NBUF - 1
            if nxt < n_ch:
                if c >= 1:
                    stores[c - 1].wait()
                gathers[nxt] = start_gather(nxt)
            gathers[c].wait()
            stores[c] = start_store(c)
        for c in range(max(0, n_ch - NBUF + 1), n_ch):
            stores[c].wait()

    return k(time2d, pe)


def kernel(time, pe):
    return _gather_rows_sc(time, pe)
